# baseline (speedup 1.0000x reference)
import functools

import jax
import jax.numpy as jnp
from jax.experimental import pallas as pl
from jax.experimental.pallas import tpu as pltpu

_VMEM_LIMIT = 64 * 1024 * 1024
_BF = jnp.bfloat16
_F32 = jnp.float32

_TA = (((0,), (0,)), ((), ()))


def _wdot(w_ref, x):
    return jax.lax.dot_general(w_ref[...], x, _TA,
                               preferred_element_type=_F32)


def _ln_cm(x, g, b, eps=1e-5):
    mu = jnp.mean(x, axis=0, keepdims=True)
    xc = x - mu
    var = jnp.mean(xc * xc, axis=0, keepdims=True)
    return (xc * jax.lax.rsqrt(var + eps)) * g + b


def _mha_cm(qt, kvt, heads, dh):
    inner = heads * dh
    outs = []
    for h in range(heads):
        lo = h * dh
        qh = qt[lo:lo + dh, :]
        kh = kvt[lo:lo + dh, :]
        vh = kvt[inner + lo:inner + lo + dh, :]
        st = jax.lax.dot_general(kh, qh, _TA,
                                 preferred_element_type=_F32)
        m = jnp.max(st, axis=0, keepdims=True)
        p = jnp.exp(st - m)
        l = jnp.sum(p, axis=0, keepdims=True)
        ot = jnp.dot(vh, p.astype(_BF), preferred_element_type=_F32)
        outs.append(ot * (1.0 / l))
    return jnp.concatenate(outs, axis=0).astype(_BF)


def _block_kernel(x_ref, ctx_ref, bias_ref, ffb_ref, w_in_ref,
                  wqkv_ref, wkv2_ref, wo_ref, wq2_ref, wo2_ref,
                  wx_ref, wg_ref, wfo_ref, wout_ref,
                  out_ref, *, groups, heads, dh):
    inner = heads * dh
    bias = bias_ref[...]
    (gng, beff, g1, b1, bo, g2, b2, bo2, g3, b3, bfo, bout) = [
        bias[:, i:i + 1] for i in range(12)]
    bx = ffb_ref[:, 0:1]
    bg = ffb_ref[:, 1:2]

    xg = x_ref[0].astype(_F32)
    C, HW = xg.shape
    xr = xg.reshape(groups, (C // groups) * HW)
    mu = jnp.mean(xr, axis=-1, keepdims=True)
    xc = xr - mu
    var = jnp.mean(xc * xc, axis=-1, keepdims=True)
    xn = (xc * jax.lax.rsqrt(var + 1e-6)).reshape(C, HW)
    xs = (xn * gng).astype(_BF)
    h = _wdot(w_in_ref, xs) + beff

    hn = _ln_cm(h, g1, b1).astype(_BF)
    qkv = _wdot(wqkv_ref, hn).astype(_BF)
    attn = _mha_cm(qkv[:inner], qkv[inner:], heads, dh)
    x2 = _wdot(wo_ref, attn) + bo + h

    kv2t = jax.lax.dot_general(wkv2_ref[...], ctx_ref[0],
                               (((0,), (1,)), ((), ())),
                               preferred_element_type=_F32).astype(_BF)
    hn2 = _ln_cm(x2, g2, b2).astype(_BF)
    q2 = _wdot(wq2_ref, hn2).astype(_BF)
    attn2 = _mha_cm(q2, kv2t, heads, dh)
    x3 = _wdot(wo2_ref, attn2) + bo2 + x2

    hn3 = _ln_cm(x3, g3, b3).astype(_BF)
    u = _wdot(wx_ref, hn3) + bx
    g = _wdot(wg_ref, hn3) + bg
    gg = (u * jax.nn.gelu(g)).astype(_BF)
    x4 = _wdot(wfo_ref, gg) + bfo + x3

    yt = jax.lax.dot_general(wout_ref[...], x4.astype(_BF), _TA,
                             preferred_element_type=_F32)
    out_ref[0] = yt + bout + xg


def kernel(x, context, gn_gamma, gn_beta, w_in, b_in, w_out, b_out,
           g1, b1, g2, b2, g3, b3,
           a1_w_qkv, a1_w_q_scaled, a1_w_kv, a1_w_o, a1_b_o,
           a2_w_q_scaled, a2_w_kv, a2_w_o, a2_b_o,
           ff_w_x, ff_w_g, ff_b_x, ff_b_g, ff_w_o, ff_b_o):
    B, C, H, W = x.shape
    HW = H * W
    heads, dh = 8, 40
    inner = heads * dh
    Lc = context.shape[1]
    Dc = context.shape[2]
    dff = ff_w_x.shape[1]

    x3d = x.reshape(B, C, HW)
    bias = jnp.stack([gn_gamma, b_in + gn_beta @ w_in, g1, b1, a1_b_o,
                      g2, b2, a2_b_o, g3, b3, ff_b_o, b_out], axis=1)
    ffb = jnp.stack([ff_b_x, ff_b_g], axis=1)
    bf = lambda w: w.astype(_BF)

    full = lambda *shape: pl.BlockSpec(shape, lambda b: (0,) * len(shape))

    out = pl.pallas_call(
        functools.partial(_block_kernel, groups=32, heads=heads, dh=dh),
        grid=(B,),
        in_specs=[
            pl.BlockSpec((1, C, HW), lambda b: (b, 0, 0)),
            pl.BlockSpec((1, Lc, Dc), lambda b: (b, 0, 0)),
            full(C, 12), full(dff, 2), full(C, inner),
            full(inner, 3 * inner), full(Dc, 2 * inner),
            full(inner, inner), full(inner, inner), full(inner, inner),
            full(inner, dff), full(inner, dff), full(dff, inner),
            full(inner, C),
        ],
        out_specs=pl.BlockSpec((1, C, HW), lambda b: (b, 0, 0)),
        out_shape=jax.ShapeDtypeStruct((B, C, HW), _F32),
        compiler_params=pltpu.CompilerParams(
            dimension_semantics=("parallel",), vmem_limit_bytes=_VMEM_LIMIT),
    )(x3d, context.astype(_BF), bias, ffb, bf(w_in),
      bf(a1_w_qkv), bf(a2_w_kv), bf(a1_w_o), bf(a2_w_q_scaled), bf(a2_w_o),
      bf(ff_w_x), bf(ff_w_g), bf(ff_w_o), bf(w_out))

    return out.reshape(B, C, H, W)


# speedup vs baseline: 1.0245x; 1.0245x over previous
import functools

import jax
import jax.numpy as jnp
from jax.experimental import pallas as pl
from jax.experimental.pallas import tpu as pltpu

_VMEM_LIMIT = 64 * 1024 * 1024
_BF = jnp.bfloat16
_F32 = jnp.float32

_TA = (((0,), (0,)), ((), ()))


def _wdot(w_ref, x):
    return jax.lax.dot_general(w_ref[...], x, _TA,
                               preferred_element_type=_F32)


def _ln_cm(x, g, b, eps=1e-5):
    mu = jnp.mean(x, axis=0, keepdims=True)
    xc = x - mu
    var = jnp.mean(xc * xc, axis=0, keepdims=True)
    return (xc * jax.lax.rsqrt(var + eps)) * g + b


def _mha_cm(qt, kvt, heads, dh):
    inner = heads * dh
    outs = []
    for h in range(heads):
        lo = h * dh
        qh = qt[lo:lo + dh, :]
        kh = kvt[lo:lo + dh, :]
        vh = kvt[inner + lo:inner + lo + dh, :]
        st = jax.lax.dot_general(kh, qh, _TA,
                                 preferred_element_type=_F32)
        m = jnp.max(st, axis=0, keepdims=True)
        p = jnp.exp(st - m)
        l = jnp.sum(p, axis=0, keepdims=True)
        ot = jnp.dot(vh, p.astype(_BF), preferred_element_type=_F32)
        outs.append(ot * (1.0 / l))
    return jnp.concatenate(outs, axis=0).astype(_BF)


def _block_kernel(x_ref, ctx_ref, bias_ref, ffb_ref, w_in_ref,
                  wqkv_ref, wkv2_ref, wo_ref, wq2_ref, wo2_ref,
                  wx_ref, wg_ref, wfo_ref, wout_ref,
                  out_ref,
                  w_in_c, wqkv_c, wkv2_c, wo_c, wq2_c, wo2_c,
                  wx_c, wg_c, wfo_c, wout_c, *, groups, heads, dh):
    inner = heads * dh

    @pl.when(pl.program_id(0) == 0)
    def _cache_weights():
        for src_ref, dst_ref in ((w_in_ref, w_in_c), (wqkv_ref, wqkv_c),
                                 (wkv2_ref, wkv2_c), (wo_ref, wo_c),
                                 (wq2_ref, wq2_c), (wo2_ref, wo2_c),
                                 (wx_ref, wx_c), (wg_ref, wg_c),
                                 (wfo_ref, wfo_c), (wout_ref, wout_c)):
            dst_ref[...] = src_ref[...].astype(_BF)

    w_in_ref, wqkv_ref, wkv2_ref, wo_ref, wq2_ref, wo2_ref = (
        w_in_c, wqkv_c, wkv2_c, wo_c, wq2_c, wo2_c)
    wx_ref, wg_ref, wfo_ref, wout_ref = wx_c, wg_c, wfo_c, wout_c
    bias = bias_ref[...]
    (gng, beff, g1, b1, bo, g2, b2, bo2, g3, b3, bfo, bout) = [
        bias[:, i:i + 1] for i in range(12)]
    bx = ffb_ref[:, 0:1]
    bg = ffb_ref[:, 1:2]

    xg = x_ref[0].astype(_F32)
    C, HW = xg.shape
    xr = xg.reshape(groups, (C // groups) * HW)
    mu = jnp.mean(xr, axis=-1, keepdims=True)
    xc = xr - mu
    var = jnp.mean(xc * xc, axis=-1, keepdims=True)
    xn = (xc * jax.lax.rsqrt(var + 1e-6)).reshape(C, HW)
    xs = (xn * gng).astype(_BF)
    h = _wdot(w_in_ref, xs) + beff

    hn = _ln_cm(h, g1, b1).astype(_BF)
    qkv = _wdot(wqkv_ref, hn).astype(_BF)
    attn = _mha_cm(qkv[:inner], qkv[inner:], heads, dh)
    x2 = _wdot(wo_ref, attn) + bo + h

    kv2t = jax.lax.dot_general(wkv2_ref[...], ctx_ref[0].astype(_BF),
                               (((0,), (1,)), ((), ())),
                               preferred_element_type=_F32).astype(_BF)
    hn2 = _ln_cm(x2, g2, b2).astype(_BF)
    q2 = _wdot(wq2_ref, hn2).astype(_BF)
    attn2 = _mha_cm(q2, kv2t, heads, dh)
    x3 = _wdot(wo2_ref, attn2) + bo2 + x2

    hn3 = _ln_cm(x3, g3, b3).astype(_BF)
    u = _wdot(wx_ref, hn3) + bx
    g = _wdot(wg_ref, hn3) + bg
    gg = (u * jax.nn.gelu(g)).astype(_BF)
    x4 = _wdot(wfo_ref, gg) + bfo + x3

    yt = jax.lax.dot_general(wout_ref[...], x4.astype(_BF), _TA,
                             preferred_element_type=_F32)
    out_ref[0] = yt + bout + xg


def kernel(x, context, gn_gamma, gn_beta, w_in, b_in, w_out, b_out,
           g1, b1, g2, b2, g3, b3,
           a1_w_qkv, a1_w_q_scaled, a1_w_kv, a1_w_o, a1_b_o,
           a2_w_q_scaled, a2_w_kv, a2_w_o, a2_b_o,
           ff_w_x, ff_w_g, ff_b_x, ff_b_g, ff_w_o, ff_b_o):
    B, C, H, W = x.shape
    HW = H * W
    heads, dh = 8, 40
    inner = heads * dh
    Lc = context.shape[1]
    Dc = context.shape[2]
    dff = ff_w_x.shape[1]

    x3d = x.reshape(B, C, HW)
    bias = jnp.stack([gn_gamma, b_in + gn_beta @ w_in, g1, b1, a1_b_o,
                      g2, b2, a2_b_o, g3, b3, ff_b_o, b_out], axis=1)
    ffb = jnp.stack([ff_b_x, ff_b_g], axis=1)

    full = lambda *shape: pl.BlockSpec(shape, lambda b: (0,) * len(shape))

    out = pl.pallas_call(
        functools.partial(_block_kernel, groups=32, heads=heads, dh=dh),
        grid=(B,),
        in_specs=[
            pl.BlockSpec((1, C, HW), lambda b: (b, 0, 0)),
            pl.BlockSpec((1, Lc, Dc), lambda b: (b, 0, 0)),
            full(C, 12), full(dff, 2), full(C, inner),
            full(inner, 3 * inner), full(Dc, 2 * inner),
            full(inner, inner), full(inner, inner), full(inner, inner),
            full(inner, dff), full(inner, dff), full(dff, inner),
            full(inner, C),
        ],
        out_specs=pl.BlockSpec((1, C, HW), lambda b: (b, 0, 0)),
        out_shape=jax.ShapeDtypeStruct((B, C, HW), _F32),
        scratch_shapes=[
            pltpu.VMEM((C, inner), _BF), pltpu.VMEM((inner, 3 * inner), _BF),
            pltpu.VMEM((Dc, 2 * inner), _BF), pltpu.VMEM((inner, inner), _BF),
            pltpu.VMEM((inner, inner), _BF), pltpu.VMEM((inner, inner), _BF),
            pltpu.VMEM((inner, dff), _BF), pltpu.VMEM((inner, dff), _BF),
            pltpu.VMEM((dff, inner), _BF), pltpu.VMEM((inner, C), _BF),
        ],
        compiler_params=pltpu.CompilerParams(
            dimension_semantics=("arbitrary",), vmem_limit_bytes=_VMEM_LIMIT),
    )(x3d, context, bias, ffb, w_in,
      a1_w_qkv, a2_w_kv, a1_w_o, a2_w_q_scaled, a2_w_o,
      ff_w_x, ff_w_g, ff_w_o, w_out)

    return out.reshape(B, C, H, W)
